# baseline (speedup 1.0000x reference)
import functools

import jax
import jax.numpy as jnp
from jax import lax
from jax.experimental import pallas as pl
from jax.experimental.pallas import tpu as pltpu
from jax.experimental.pallas import tpu_sc as plsc

N = 10000
E = 160000
D = 256
NC = 2
NS = 16
L = 16
NW = NC * NS

E_PAD = 163840
EPT = E_PAD // NW
EPS = E_PAD // NS
N_PAD = 10240
HALF = N // 2
RPT = 320
HALF_PAD = NS * RPT
ACC_ROWS = HALF_PAD + 16
CH = 128
EB = 2560

_mesh = plsc.VectorSubcoreMesh(core_axis_name="c", subcore_axis_name="s")


def _iota16():
    return lax.broadcasted_iota(jnp.int32, (L,), 0)


def _deg_body(dst_hbm, degp_hbm, dst_v, deg_loc, sem):
    c = lax.axis_index("c")
    s = lax.axis_index("s")
    w = s * NC + c

    def zero(g, _):
        deg_loc[pl.ds(g * L, L)] = jnp.zeros((L,), jnp.float32)
        return 0

    lax.fori_loop(0, N_PAD // L, zero, 0)

    pltpu.async_copy(dst_hbm.at[pl.ds(w * EPT, EPT)], dst_v, sem).wait()

    ones = jnp.ones((L,), jnp.float32)

    def upd(g, _):
        d = dst_v[pl.ds(g * L, L)]
        plsc.addupdate_scatter(deg_loc, [d], ones)
        return 0

    lax.fori_loop(0, EPT // L, upd, 0)
    pltpu.sync_copy(deg_loc, degp_hbm.at[w])


_deg_call = pl.kernel(
    _deg_body,
    out_type=jax.ShapeDtypeStruct((NW, N_PAD), jnp.float32),
    mesh=_mesh,
    compiler_params=pltpu.CompilerParams(needs_layout_passes=False),
    scratch_types=[
        pltpu.VMEM((EPT,), jnp.int32),
        pltpu.VMEM((N_PAD,), jnp.float32),
        pltpu.SemaphoreType.DMA,
    ],
)


def _xw_body(z_ref, w_ref, degp_ref, xws_ref, dis_ref):
    i = pl.program_id(0)
    deg = jnp.sum(degp_ref[...], axis=0) + 1.0
    dis = lax.rsqrt(deg)
    xw = jnp.dot(z_ref[...], w_ref[...],
                 preferred_element_type=jnp.float32,
                 precision=lax.Precision.HIGHEST)
    xws_ref[...] = dis[:, None] * xw
    dis_ref[pl.ds(i * 2048, 2048)] = dis


def _xw_kernel(z_pad, W, degp):
    return pl.pallas_call(
        _xw_body,
        grid=(5,),
        in_specs=[
            pl.BlockSpec((2048, D), lambda i: (i, 0)),
            pl.BlockSpec((D, D), lambda i: (0, 0)),
            pl.BlockSpec((NW, 2048), lambda i: (0, i)),
        ],
        out_specs=[
            pl.BlockSpec((2048, D), lambda i: (i, 0)),
            pl.BlockSpec((N_PAD,), lambda i: (0,)),
        ],
        out_shape=[
            jax.ShapeDtypeStruct((N_PAD, D), jnp.float32),
            jax.ShapeDtypeStruct((N_PAD,), jnp.float32),
        ],
    )(z_pad, W, degp)


RPT_N = N_PAD // NW
ACC_R = RPT_N + 8
EB3 = 2048
LCAP = EB3 + CH


def _agg_body(xws_hbm, src_hbm, dst_hbm, acc_hbm,
              src_v, dst_v, li_src, li_dst, rows_v, acc_v, sem):
    c = lax.axis_index("c")
    s = lax.axis_index("s")
    w = s * NC + c
    lo = w * RPT_N

    pltpu.sync_copy(xws_hbm.at[pl.ds(lo, RPT_N)], acc_v.at[pl.ds(0, RPT_N)])

    iota = _iota16()
    dummy = jnp.full((L,), RPT_N, jnp.int32)

    def block(nb, _):
        ebase = nb * EB3
        pltpu.async_copy(src_hbm.at[pl.ds(ebase, EB3)], src_v, sem).wait()
        pltpu.async_copy(dst_hbm.at[pl.ds(ebase, EB3)], dst_v, sem).wait()

        def compress(g, cnt):
            sv = src_v[pl.ds(g * L, L)]
            dv = dst_v[pl.ds(g * L, L)]
            local = dv - lo
            ok = (local >= 0) & (local < RPT_N)
            plsc.store_compressed(li_src.at[pl.ds(cnt, L)], sv, mask=ok)
            plsc.store_compressed(li_dst.at[pl.ds(cnt, L)], local, mask=ok)
            return cnt + jnp.sum(ok.astype(jnp.int32))

        cnt = lax.fori_loop(0, EB3 // L, compress, 0)

        for p8 in range(CH // L):
            li_src[pl.ds(cnt + p8 * L, L)] = jnp.zeros((L,), jnp.int32)
            li_dst[pl.ds(cnt + p8 * L, L)] = dummy

        def chunk(j, _):
            pltpu.async_copy(
                xws_hbm.at[li_src.at[pl.ds(j * CH, CH)]], rows_v, sem).wait()
            for g8 in range(CH // L):
                dloc = li_dst[pl.ds(j * CH + g8 * L, L)]
                ridx = g8 * L + iota

                def feat(ph, _):
                    for pl_ in range(L):
                        p = ph * L + pl_
                        pv = jnp.full((L,), p, jnp.int32)
                        val = plsc.load_gather(rows_v, [ridx, pv])
                        plsc.addupdate_scatter(acc_v, [dloc, pv], val)
                    return 0

                lax.fori_loop(0, D // L, feat, 0)
            return 0

        lax.fori_loop(0, (cnt + CH - 1) // CH, chunk, 0)
        return 0

    lax.fori_loop(0, E_PAD // EB3, block, 0)

    pltpu.sync_copy(acc_v.at[pl.ds(0, RPT_N)], acc_hbm.at[pl.ds(lo, RPT_N)])


_agg_call = pl.kernel(
    _agg_body,
    out_type=jax.ShapeDtypeStruct((N_PAD, D), jnp.float32),
    mesh=_mesh,
    compiler_params=pltpu.CompilerParams(needs_layout_passes=False),
    scratch_types=[
        pltpu.VMEM((EB3,), jnp.int32),
        pltpu.VMEM((EB3,), jnp.int32),
        pltpu.VMEM((LCAP,), jnp.int32),
        pltpu.VMEM((LCAP,), jnp.int32),
        pltpu.VMEM((CH, D), jnp.float32),
        pltpu.VMEM((ACC_R, D), jnp.float32),
        pltpu.SemaphoreType.DMA,
    ],
)


def _dot_body(z_hbm, src_hbm, dst_hbm, val_hbm,
              src_v, dst_v, za, zb, val_v, sem):
    c = lax.axis_index("c")
    s = lax.axis_index("s")
    w = s * NC + c

    pltpu.async_copy(src_hbm.at[pl.ds(w * EPT, EPT)], src_v, sem).wait()
    pltpu.async_copy(dst_hbm.at[pl.ds(w * EPT, EPT)], dst_v, sem).wait()

    lane0 = _iota16() == 0

    def chunk(j, _):
        base = j * CH
        pltpu.async_copy(z_hbm.at[src_v.at[pl.ds(base, CH)]], za, sem).wait()
        pltpu.async_copy(z_hbm.at[dst_v.at[pl.ds(base, CH)]], zb, sem).wait()

        def row(r, _):
            acc = jnp.zeros((L,), jnp.float32)
            for k in range(D // L):
                acc = acc + za[r, pl.ds(k * L, L)] * zb[r, pl.ds(k * L, L)]
            dot = jnp.sum(acc)
            plsc.store_scatter(val_v, [jnp.full((L,), base + r, jnp.int32)],
                               jnp.full((L,), dot, jnp.float32), mask=lane0)
            return 0

        lax.fori_loop(0, CH, row, 0)
        return 0

    lax.fori_loop(0, EPT // CH, chunk, 0)

    def sig(g, _):
        v = val_v[pl.ds(g * L, L)]
        val_v[pl.ds(g * L, L)] = 1.0 / (1.0 + jnp.exp(-v))
        return 0

    lax.fori_loop(0, EPT // L, sig, 0)
    pltpu.sync_copy(val_v, val_hbm.at[pl.ds(w * EPT, EPT)])


_dot_call = pl.kernel(
    _dot_body,
    out_type=jax.ShapeDtypeStruct((E_PAD,), jnp.float32),
    mesh=_mesh,
    compiler_params=pltpu.CompilerParams(needs_layout_passes=False),
    scratch_types=[
        pltpu.VMEM((EPT,), jnp.int32),
        pltpu.VMEM((EPT,), jnp.int32),
        pltpu.VMEM((CH, D), jnp.float32),
        pltpu.VMEM((CH, D), jnp.float32),
        pltpu.VMEM((EPT,), jnp.float32),
        pltpu.SemaphoreType.DMA,
    ],
)


def _out_body(acc_ref, dis_ref, b_ref, out_ref):
    i = pl.program_id(0)
    dis = dis_ref[pl.ds(i * 1024, 1024)]
    out_ref[...] = dis[:, None] * acc_ref[...] + b_ref[...][None, :]


def _out_kernel(acc, dis, b):
    return pl.pallas_call(
        _out_body,
        grid=(10,),
        in_specs=[
            pl.BlockSpec((1024, D), lambda i: (i, 0)),
            pl.BlockSpec((N_PAD,), lambda i: (0,)),
            pl.BlockSpec((D,), lambda i: (0,)),
        ],
        out_specs=pl.BlockSpec((1024, D), lambda i: (i, 0)),
        out_shape=jax.ShapeDtypeStruct((N, D), jnp.float32),
    )(acc, dis, b)


def kernel(z, edge_index_t, W, b):
    src = edge_index_t[0]
    dst = edge_index_t[1]
    pad = E_PAD - E
    src_pad = jnp.concatenate([src, jnp.zeros((pad,), jnp.int32)])
    dst_pad = jnp.concatenate([dst, jnp.full((pad,), N, jnp.int32)])

    z_pad = jnp.pad(z, ((0, N_PAD - N), (0, 0)))
    degp = _deg_call(dst_pad)
    xws, dis = _xw_kernel(z_pad, W, degp)
    acc = _agg_call(xws, src_pad, dst_pad)
    val_pad = _dot_call(z, src_pad, dst_pad)
    z_dec = _out_kernel(acc, dis, b)
    return (z_dec, val_pad[:E])


# speedup vs baseline: 6.4843x; 6.4843x over previous
import functools

import jax
import jax.numpy as jnp
from jax import lax
from jax.experimental import pallas as pl
from jax.experimental.pallas import tpu as pltpu
from jax.experimental.pallas import tpu_sc as plsc

N = 10000
E = 160000
D = 256
NC = 2
NS = 16
L = 16
NW = NC * NS

E_PAD = 163840
EPT = E_PAD // NW
EPS = E_PAD // NS
N_PAD = 10240
HALF = N // 2
RPT = 320
HALF_PAD = NS * RPT
ACC_ROWS = HALF_PAD + 16
CH = 128
EB = 2560

_mesh = plsc.VectorSubcoreMesh(core_axis_name="c", subcore_axis_name="s")


def _iota16():
    return lax.broadcasted_iota(jnp.int32, (L,), 0)


def _deg_body(dst_hbm, degp_hbm, dst_v, deg_loc, sem):
    c = lax.axis_index("c")
    s = lax.axis_index("s")
    w = s * NC + c

    def zero(g, _):
        deg_loc[pl.ds(g * L, L)] = jnp.zeros((L,), jnp.float32)
        return 0

    lax.fori_loop(0, N_PAD // L, zero, 0)

    pltpu.async_copy(dst_hbm.at[pl.ds(w * EPT, EPT)], dst_v, sem).wait()

    ones = jnp.ones((L,), jnp.float32)

    def upd(g, _):
        d = dst_v[pl.ds(g * L, L)]
        plsc.addupdate_scatter(deg_loc, [d], ones)
        return 0

    lax.fori_loop(0, EPT // L, upd, 0)
    pltpu.sync_copy(deg_loc, degp_hbm.at[w])


_deg_call = pl.kernel(
    _deg_body,
    out_type=jax.ShapeDtypeStruct((NW, N_PAD), jnp.float32),
    mesh=_mesh,
    compiler_params=pltpu.CompilerParams(needs_layout_passes=False),
    scratch_types=[
        pltpu.VMEM((EPT,), jnp.int32),
        pltpu.VMEM((N_PAD,), jnp.float32),
        pltpu.SemaphoreType.DMA,
    ],
)


def _xw_body(z_ref, w_ref, degp_ref, xws_ref, dis_ref):
    i = pl.program_id(0)
    deg = jnp.sum(degp_ref[...], axis=0) + 1.0
    dis = lax.rsqrt(deg)
    xw = jnp.dot(z_ref[...], w_ref[...],
                 preferred_element_type=jnp.float32,
                 precision=lax.Precision.HIGHEST)
    xws_ref[...] = dis[:, None] * xw
    dis_ref[pl.ds(i * 2048, 2048)] = dis


def _xw_kernel(z_pad, W, degp):
    return pl.pallas_call(
        _xw_body,
        grid=(5,),
        in_specs=[
            pl.BlockSpec((2048, D), lambda i: (i, 0)),
            pl.BlockSpec((D, D), lambda i: (0, 0)),
            pl.BlockSpec((NW, 2048), lambda i: (0, i)),
        ],
        out_specs=[
            pl.BlockSpec((2048, D), lambda i: (i, 0)),
            pl.BlockSpec((N_PAD,), lambda i: (0,)),
        ],
        out_shape=[
            jax.ShapeDtypeStruct((N_PAD, D), jnp.float32),
            jax.ShapeDtypeStruct((N_PAD,), jnp.float32),
        ],
    )(z_pad, W, degp)


RPT_N = N_PAD // NW
ACC_R = RPT_N + 8
EB3 = 2048
NBLK = E_PAD // EB3
LCAP = EB3 + 2 * CH


def _agg_body(xws_hbm, src_hbm, dst_hbm, acc_hbm,
              src_a, dst_a, src_b, dst_b, li_src, li_dst, rows_v, acc_v,
              sem_a, sem_b, sem_g):
    c = lax.axis_index("c")
    s = lax.axis_index("s")
    w = s * NC + c
    lo = w * RPT_N

    pltpu.sync_copy(xws_hbm.at[pl.ds(lo, RPT_N)], acc_v.at[pl.ds(0, RPT_N)])

    def fetch(nb, sv, dv, sem):
        pltpu.async_copy(src_hbm.at[pl.ds(nb * EB3, EB3)], sv, sem)
        pltpu.async_copy(dst_hbm.at[pl.ds(nb * EB3, EB3)], dv, sem)

    def fetch_wait(sv, dv, sem):
        pltpu.make_async_copy(src_hbm.at[pl.ds(0, EB3)], sv, sem).wait()
        pltpu.make_async_copy(dst_hbm.at[pl.ds(0, EB3)], dv, sem).wait()

    def accum_chunk(j):
        pltpu.async_copy(
            xws_hbm.at[li_src.at[pl.ds(j * CH, CH)]], rows_v, sem_g).wait()

        def rowgrp(g8, _):
            dvec = li_dst[pl.ds(j * CH + g8 * L, L)]
            for rr in range(L):
                dloc = dvec[rr]
                r = g8 * L + rr
                for k in range(D // L):
                    plsc.addupdate(acc_v.at[dloc, pl.ds(k * L, L)],
                                   rows_v[r, pl.ds(k * L, L)])
            return 0

        lax.fori_loop(0, CH // L, rowgrp, 0)

    def process(sv, dv, rem):
        def compress(g, cnt):
            svv = sv[pl.ds(g * L, L)]
            dvv = dv[pl.ds(g * L, L)]
            local = dvv - lo
            ok = (local >= 0) & (local < RPT_N)
            plsc.store_compressed(li_src.at[pl.ds(cnt, L)], svv, mask=ok)
            plsc.store_compressed(li_dst.at[pl.ds(cnt, L)], local, mask=ok)
            return cnt + jnp.sum(ok.astype(jnp.int32))

        cnt = lax.fori_loop(0, EB3 // L, compress, rem)
        nch = cnt // CH

        def chunk(j, _):
            accum_chunk(j)
            return 0

        lax.fori_loop(0, nch, chunk, 0)

        for g in range(CH // L):
            li_src[pl.ds(g * L, L)] = li_src[pl.ds(nch * CH + g * L, L)]
            li_dst[pl.ds(g * L, L)] = li_dst[pl.ds(nch * CH + g * L, L)]
        return cnt - nch * CH

    fetch(0, src_a, dst_a, sem_a)

    def pair(m, rem):
        fetch_wait(src_a, dst_a, sem_a)
        fetch(2 * m + 1, src_b, dst_b, sem_b)
        rem = process(src_a, dst_a, rem)
        fetch_wait(src_b, dst_b, sem_b)

        @pl.when(m < NBLK // 2 - 1)
        def _():
            fetch(2 * m + 2, src_a, dst_a, sem_a)

        return process(src_b, dst_b, rem)

    rem = lax.fori_loop(0, NBLK // 2, pair, 0)

    dummy = jnp.full((L,), RPT_N, jnp.int32)
    for p8 in range(CH // L):
        li_src[pl.ds(rem + p8 * L, L)] = jnp.zeros((L,), jnp.int32)
        li_dst[pl.ds(rem + p8 * L, L)] = dummy
    accum_chunk(0)

    pltpu.sync_copy(acc_v.at[pl.ds(0, RPT_N)], acc_hbm.at[pl.ds(lo, RPT_N)])


_agg_call = pl.kernel(
    _agg_body,
    out_type=jax.ShapeDtypeStruct((N_PAD, D), jnp.float32),
    mesh=_mesh,
    compiler_params=pltpu.CompilerParams(needs_layout_passes=False),
    scratch_types=[
        pltpu.VMEM((EB3,), jnp.int32),
        pltpu.VMEM((EB3,), jnp.int32),
        pltpu.VMEM((EB3,), jnp.int32),
        pltpu.VMEM((EB3,), jnp.int32),
        pltpu.VMEM((LCAP,), jnp.int32),
        pltpu.VMEM((LCAP,), jnp.int32),
        pltpu.VMEM((CH, D), jnp.float32),
        pltpu.VMEM((ACC_R, D), jnp.float32),
        pltpu.SemaphoreType.DMA,
        pltpu.SemaphoreType.DMA,
        pltpu.SemaphoreType.DMA,
    ],
)


def _dot_body(z_hbm, src_hbm, dst_hbm, val_hbm,
              src_v, dst_v, za, zb, val_v, sem):
    c = lax.axis_index("c")
    s = lax.axis_index("s")
    w = s * NC + c

    pltpu.async_copy(src_hbm.at[pl.ds(w * EPT, EPT)], src_v, sem).wait()
    pltpu.async_copy(dst_hbm.at[pl.ds(w * EPT, EPT)], dst_v, sem).wait()

    lane0 = _iota16() == 0

    def chunk(j, _):
        base = j * CH
        pltpu.async_copy(z_hbm.at[src_v.at[pl.ds(base, CH)]], za, sem).wait()
        pltpu.async_copy(z_hbm.at[dst_v.at[pl.ds(base, CH)]], zb, sem).wait()

        def row(r, _):
            acc = jnp.zeros((L,), jnp.float32)
            for k in range(D // L):
                acc = acc + za[r, pl.ds(k * L, L)] * zb[r, pl.ds(k * L, L)]
            dot = jnp.sum(acc)
            plsc.store_scatter(val_v, [jnp.full((L,), base + r, jnp.int32)],
                               jnp.full((L,), dot, jnp.float32), mask=lane0)
            return 0

        lax.fori_loop(0, CH, row, 0)
        return 0

    lax.fori_loop(0, EPT // CH, chunk, 0)

    def sig(g, _):
        v = val_v[pl.ds(g * L, L)]
        val_v[pl.ds(g * L, L)] = 1.0 / (1.0 + jnp.exp(-v))
        return 0

    lax.fori_loop(0, EPT // L, sig, 0)
    pltpu.sync_copy(val_v, val_hbm.at[pl.ds(w * EPT, EPT)])


_dot_call = pl.kernel(
    _dot_body,
    out_type=jax.ShapeDtypeStruct((E_PAD,), jnp.float32),
    mesh=_mesh,
    compiler_params=pltpu.CompilerParams(needs_layout_passes=False),
    scratch_types=[
        pltpu.VMEM((EPT,), jnp.int32),
        pltpu.VMEM((EPT,), jnp.int32),
        pltpu.VMEM((CH, D), jnp.float32),
        pltpu.VMEM((CH, D), jnp.float32),
        pltpu.VMEM((EPT,), jnp.float32),
        pltpu.SemaphoreType.DMA,
    ],
)


def _out_body(acc_ref, dis_ref, b_ref, out_ref):
    i = pl.program_id(0)
    dis = dis_ref[pl.ds(i * 1024, 1024)]
    out_ref[...] = dis[:, None] * acc_ref[...] + b_ref[...][None, :]


def _out_kernel(acc, dis, b):
    return pl.pallas_call(
        _out_body,
        grid=(10,),
        in_specs=[
            pl.BlockSpec((1024, D), lambda i: (i, 0)),
            pl.BlockSpec((N_PAD,), lambda i: (0,)),
            pl.BlockSpec((D,), lambda i: (0,)),
        ],
        out_specs=pl.BlockSpec((1024, D), lambda i: (i, 0)),
        out_shape=jax.ShapeDtypeStruct((N, D), jnp.float32),
    )(acc, dis, b)


def kernel(z, edge_index_t, W, b):
    src = edge_index_t[0]
    dst = edge_index_t[1]
    pad = E_PAD - E
    src_pad = jnp.concatenate([src, jnp.zeros((pad,), jnp.int32)])
    dst_pad = jnp.concatenate([dst, jnp.full((pad,), N, jnp.int32)])

    z_pad = jnp.pad(z, ((0, N_PAD - N), (0, 0)))
    degp = _deg_call(dst_pad)
    xws, dis = _xw_kernel(z_pad, W, degp)
    acc = _agg_call(xws, src_pad, dst_pad)
    val_pad = _dot_call(z, src_pad, dst_pad)
    z_dec = _out_kernel(acc, dis, b)
    return (z_dec, val_pad[:E])
